# baseline (speedup 1.0000x reference)
import jax
import jax.numpy as jnp
import numpy as np
from jax import lax
from jax.experimental import pallas as pl
from jax.experimental.pallas import tpu as pltpu
from jax.experimental.pallas import tpu_sc as plsc

B = 50000
D = 128
E = 128
NSAMP = 10
NLSH = 5

NC = 2
NS = 16
NW = NC * NS
C = 56
K0 = 17
K1 = 11
KMAX = max(K0, K1)
W0 = C * K0
W1 = C * K1
PWMAX = C * KMAX
BP = NS * (W0 + W1)
BPX = BP + C * (K0 + K1)

TB = 3584


def _sc_body_fn(feat, nodes, neigh, lsh, comb_o,
             nodes_v, neigh_v, lsh_v, srows, nacc, lacc, stage,
             gsem, asem, ssem):
  cid = lax.axis_index("c")
  sid = lax.axis_index("s")
  my_base = lax.select(cid == 0, sid * W0, NS * W0 + sid * W1)
  my_nchunk = lax.select(cid == 0, jnp.int32(K0), jnp.int32(K1))
  pltpu.sync_copy(nodes.at[pl.ds(my_base, PWMAX)], nodes_v)
  for r in range(NSAMP):
    pltpu.sync_copy(neigh.at[pl.ds(r * BPX + my_base, PWMAX)],
                    neigh_v.at[pl.ds(r * PWMAX, PWMAX)])
  for r in range(NLSH):
    pltpu.sync_copy(lsh.at[pl.ds(r * BPX + my_base, PWMAX)],
                    lsh_v.at[pl.ds(r * PWMAX, PWMAX)])

  def base_descs(c, b):
    return (
        pltpu.make_async_copy(
            feat.at[nodes_v.at[pl.ds(c * C, C)]], srows.at[b], gsem.at[b]),
        pltpu.make_async_copy(
            feat.at[neigh_v.at[pl.ds(c * C, C)]], nacc.at[b], gsem.at[b]),
        pltpu.make_async_copy(
            feat.at[lsh_v.at[pl.ds(c * C, C)]], lacc.at[b], gsem.at[b]),
    )

  def issue_adds(c, b):
    for r in range(1, NSAMP):
      pltpu.async_copy(
          feat.at[neigh_v.at[pl.ds(r * PWMAX + c * C, C)]], nacc.at[b],
          asem.at[b], add=True)
    for r in range(1, NLSH):
      pltpu.async_copy(
          feat.at[lsh_v.at[pl.ds(r * PWMAX + c * C, C)]], lacc.at[b],
          asem.at[b], add=True)

  def wait_adds(c, b):
    d = pltpu.make_async_copy(
        feat.at[neigh_v.at[pl.ds(c * C, C)]], nacc.at[b], asem.at[b])
    for _ in range(NSAMP - 1 + NLSH - 1):
      d.wait()

  def scatter_desc(c, b):
    base = my_base + c * C
    return pltpu.make_async_copy(
        stage.at[b], comb_o.at[pl.ds(base, C)], ssem.at[b])

  for d in base_descs(0, 0):
    d.start()
  for d in base_descs(1, 1):
    d.start()
  for d in base_descs(0, 0):
    d.wait()
  issue_adds(0, 0)

  def body(c, carry):
    b = lax.rem(c, 2)
    nb = 1 - b

    @pl.when(c + 1 < my_nchunk)
    def _():
      for d in base_descs(c + 1, nb):
        d.wait()
      issue_adds(c + 1, nb)

    wait_adds(c, b)

    @pl.when(c >= 2)
    def _():
      scatter_desc(c, b).wait()

    for e in range(C):
      for j in range(D // 16):
        sl = pl.ds(j * 16, 16)
        stage[b, e, sl] = srows[b, e, sl]
        stage[b, e, pl.ds(D + j * 16, 16)] = (
            nacc[b, e, sl] * jnp.float32(1.0 / NSAMP))
        stage[b, e, pl.ds(2 * D + j * 16, 16)] = (
            lacc[b, e, sl] * jnp.float32(1.0 / NLSH))

    scatter_desc(c, b).start()

    @pl.when(c + 2 < my_nchunk)
    def _():
      for d in base_descs(c + 2, b):
        d.start()

    return carry

  lax.fori_loop(0, my_nchunk, body, 0)

  scatter_desc(my_nchunk - 2, 0).wait()
  scatter_desc(my_nchunk - 1, 1).wait()


_sc_gather = pl.kernel(
    _sc_body_fn,
    out_type=jax.ShapeDtypeStruct((BP, 3 * D), jnp.float32),
    mesh=plsc.VectorSubcoreMesh(
        core_axis_name="c", subcore_axis_name="s",
        num_cores=NC, num_subcores=NS),
    scratch_types=[
        pltpu.VMEM((PWMAX,), jnp.int32),
        pltpu.VMEM((PWMAX * NSAMP,), jnp.int32),
        pltpu.VMEM((PWMAX * NLSH,), jnp.int32),
        pltpu.VMEM((2, C, D), jnp.float32),
        pltpu.VMEM((2, C, D), jnp.float32),
        pltpu.VMEM((2, C, D), jnp.float32),
        pltpu.VMEM((2, C, 3 * D), jnp.float32),
        pltpu.SemaphoreType.DMA((2,)),
        pltpu.SemaphoreType.DMA((2,)),
        pltpu.SemaphoreType.DMA((2,)),
    ],
)


def _mm_body(w_ref, x_ref, o_ref):
  y = lax.dot_general(w_ref[...], x_ref[...], (((1,), (1,)), ((), ())),
                      preferred_element_type=jnp.float32)
  o_ref[...] = jnp.maximum(y, 0.0)


_matmul = pl.pallas_call(
    _mm_body,
    grid=(BP // TB,),
    in_specs=[
        pl.BlockSpec((E, 3 * D), lambda i: (0, 0)),
        pl.BlockSpec((TB, 3 * D), lambda i: (i, 0)),
    ],
    out_specs=pl.BlockSpec((E, TB), lambda i: (0, i)),
    out_shape=jax.ShapeDtypeStruct((E, BP), jnp.float32),
)


@jax.jit
def kernel(nodes, neigh_idx, lsh_idx, features, W):
  pad = 2 * BPX - B
  nodes_f = jnp.pad(nodes, (0, pad))
  neigh_f = jnp.pad(neigh_idx, ((0, pad), (0, 0)))
  lsh_f = jnp.pad(lsh_idx, ((0, pad), (0, 0)))
  outs = []
  for h in range(2):
    sl = slice(h * BP, h * BP + BPX)
    comb = _sc_gather(features, nodes_f[sl],
                      neigh_f[sl].T.reshape(-1), lsh_f[sl].T.reshape(-1))
    outs.append(_matmul(W, comb))
  return jnp.concatenate(outs, axis=1)[:, :B]


# speedup vs baseline: 1.2057x; 1.2057x over previous
import jax
import jax.numpy as jnp
import numpy as np
from jax import lax
from jax.experimental import pallas as pl
from jax.experimental.pallas import tpu as pltpu
from jax.experimental.pallas import tpu_sc as plsc

B = 50000
D = 128
E = 128
NSAMP = 10
NLSH = 5

NC = 2
NS = 16
NW = NC * NS
C = 56
K0 = 34
K1 = 22
KMAX = max(K0, K1)
W0 = C * K0
W1 = C * K1
PWMAX = C * KMAX
BP = NS * (W0 + W1)
BPX = BP + C * (K0 + K1)

TB = 3584


def _sc_body(feat, nodes, neigh, lsh, comb_o,
             nodes_v, neigh_v, lsh_v, srows, nacc, lacc, stage,
             gsem, asem, ssem):
  cid = lax.axis_index("c")
  sid = lax.axis_index("s")
  my_base = lax.select(cid == 0, sid * W0, NS * W0 + sid * W1)
  my_nchunk = lax.select(cid == 0, jnp.int32(K0), jnp.int32(K1))
  pltpu.sync_copy(nodes.at[pl.ds(my_base, PWMAX)], nodes_v)
  for r in range(NSAMP):
    pltpu.sync_copy(neigh.at[pl.ds(r * BPX + my_base, PWMAX)],
                    neigh_v.at[pl.ds(r * PWMAX, PWMAX)])
  for r in range(NLSH):
    pltpu.sync_copy(lsh.at[pl.ds(r * BPX + my_base, PWMAX)],
                    lsh_v.at[pl.ds(r * PWMAX, PWMAX)])

  def base_descs(c, b):
    return (
        pltpu.make_async_copy(
            feat.at[nodes_v.at[pl.ds(c * C, C)]], srows.at[b], gsem.at[b]),
        pltpu.make_async_copy(
            feat.at[neigh_v.at[pl.ds(c * C, C)]], nacc.at[b], gsem.at[b]),
        pltpu.make_async_copy(
            feat.at[lsh_v.at[pl.ds(c * C, C)]], lacc.at[b], gsem.at[b]),
    )

  def issue_adds(c, b):
    for r in range(1, NSAMP):
      pltpu.async_copy(
          feat.at[neigh_v.at[pl.ds(r * PWMAX + c * C, C)]], nacc.at[b],
          asem.at[b], add=True)
    for r in range(1, NLSH):
      pltpu.async_copy(
          feat.at[lsh_v.at[pl.ds(r * PWMAX + c * C, C)]], lacc.at[b],
          asem.at[b], add=True)

  def wait_adds(c, b):
    d = pltpu.make_async_copy(
        feat.at[neigh_v.at[pl.ds(c * C, C)]], nacc.at[b], asem.at[b])
    for _ in range(NSAMP - 1 + NLSH - 1):
      d.wait()

  def scatter_desc(c, b):
    base = my_base + c * C
    return pltpu.make_async_copy(
        stage.at[b], comb_o.at[pl.ds(base, C)], ssem.at[b])

  for d in base_descs(0, 0):
    d.start()
  for d in base_descs(1, 1):
    d.start()
  for d in base_descs(0, 0):
    d.wait()
  issue_adds(0, 0)

  def body(c, carry):
    b = lax.rem(c, 2)
    nb = 1 - b

    @pl.when(c + 1 < my_nchunk)
    def _():
      for d in base_descs(c + 1, nb):
        d.wait()
      issue_adds(c + 1, nb)

    wait_adds(c, b)

    @pl.when(c >= 2)
    def _():
      scatter_desc(c, b).wait()

    for e in range(C):
      for j in range(D // 16):
        sl = pl.ds(j * 16, 16)
        stage[b, e, sl] = srows[b, e, sl]
        stage[b, e, pl.ds(D + j * 16, 16)] = (
            nacc[b, e, sl] * jnp.float32(1.0 / NSAMP))
        stage[b, e, pl.ds(2 * D + j * 16, 16)] = (
            lacc[b, e, sl] * jnp.float32(1.0 / NLSH))

    scatter_desc(c, b).start()

    @pl.when(c + 2 < my_nchunk)
    def _():
      for d in base_descs(c + 2, b):
        d.start()

    return carry

  lax.fori_loop(0, my_nchunk, body, 0)

  scatter_desc(my_nchunk - 2, 0).wait()
  scatter_desc(my_nchunk - 1, 1).wait()


_sc_gather = pl.kernel(
    _sc_body,
    out_type=jax.ShapeDtypeStruct((BP, 3 * D), jnp.float32),
    mesh=plsc.VectorSubcoreMesh(
        core_axis_name="c", subcore_axis_name="s",
        num_cores=NC, num_subcores=NS),
    scratch_types=[
        pltpu.VMEM((PWMAX,), jnp.int32),
        pltpu.VMEM((PWMAX * NSAMP,), jnp.int32),
        pltpu.VMEM((PWMAX * NLSH,), jnp.int32),
        pltpu.VMEM((2, C, D), jnp.float32),
        pltpu.VMEM((2, C, D), jnp.float32),
        pltpu.VMEM((2, C, D), jnp.float32),
        pltpu.VMEM((2, C, 3 * D), jnp.float32),
        pltpu.SemaphoreType.DMA((2,)),
        pltpu.SemaphoreType.DMA((2,)),
        pltpu.SemaphoreType.DMA((2,)),
    ],
)


def _mm_body(w_ref, x_ref, o_ref):
  y = lax.dot_general(w_ref[...], x_ref[...], (((1,), (1,)), ((), ())),
                      preferred_element_type=jnp.float32)
  o_ref[...] = jnp.maximum(y, 0.0)


_matmul = pl.pallas_call(
    _mm_body,
    grid=(BP // TB,),
    in_specs=[
        pl.BlockSpec((E, 3 * D), lambda i: (0, 0)),
        pl.BlockSpec((TB, 3 * D), lambda i: (i, 0)),
    ],
    out_specs=pl.BlockSpec((E, TB), lambda i: (0, i)),
    out_shape=jax.ShapeDtypeStruct((E, B), jnp.float32),
)


@jax.jit
def kernel(nodes, neigh_idx, lsh_idx, features, W):
  pad = BPX - B
  nodes_p = jnp.pad(nodes, (0, pad))
  neigh_p = jnp.pad(neigh_idx, ((0, pad), (0, 0))).T.reshape(-1)
  lsh_p = jnp.pad(lsh_idx, ((0, pad), (0, 0))).T.reshape(-1)
  combined = _sc_gather(features, nodes_p, neigh_p, lsh_p)
  return _matmul(W, combined)
